# baseline (speedup 1.0000x reference)
import functools

import jax
import jax.numpy as jnp
from jax.experimental import pallas as pl
from jax.experimental.pallas import tpu as pltpu

_ALPHA = 0.2
_PREC = jax.lax.Precision.DEFAULT


def _main_kernel(adj_ref, x_ref, w_ref, b_ref, mu_ref, z_ref, q_ref,
                 sup_ref, *, bm):
    i = pl.program_id(0)

    @pl.when(i == 0)
    def _():
        sup_ref[...] = jnp.dot(x_ref[...], w_ref[...],
                               preferred_element_type=jnp.float32,
                               precision=_PREC)

    z = jnp.dot(adj_ref[...], sup_ref[...],
                preferred_element_type=jnp.float32,
                precision=_PREC) + b_ref[...]
    z_ref[pl.ds(i * bm, bm), :] = z
    mu = mu_ref[...]
    zsq = jnp.sum(z * z, axis=1, keepdims=True)
    musq = jnp.sum(mu * mu, axis=1)
    cross = jax.lax.dot_general(
        z, mu, dimension_numbers=(((1,), (1,)), ((), ())),
        preferred_element_type=jnp.float32, precision=_PREC)
    d2 = zsq + musq[None, :] - 2.0 * cross
    q = 1.0 / (1.0 + d2 / _ALPHA + 1e-8)
    q = q ** (_ALPHA + 1.0)
    q_ref[pl.ds(i * bm, bm), :] = q / jnp.sum(q, axis=1, keepdims=True)


def kernel(x, adj, W, b, mu):
    n, nfeat = x.shape
    nhid = W.shape[1]
    nclust = mu.shape[0]

    bm = 400
    z, q = pl.pallas_call(
        functools.partial(_main_kernel, bm=bm),
        grid=(n // bm,),
        in_specs=[
            pl.BlockSpec((bm, n), lambda i: (i, 0)),
            pl.BlockSpec((n, nfeat), lambda i: (0, 0)),
            pl.BlockSpec((nfeat, nhid), lambda i: (0, 0)),
            pl.BlockSpec((1, nhid), lambda i: (0, 0)),
            pl.BlockSpec((nclust, nhid), lambda i: (0, 0)),
        ],
        out_specs=[
            pl.BlockSpec((n, nhid), lambda i: (0, 0)),
            pl.BlockSpec((n, nclust), lambda i: (0, 0)),
        ],
        out_shape=[
            jax.ShapeDtypeStruct((n, nhid), jnp.float32),
            jax.ShapeDtypeStruct((n, nclust), jnp.float32),
        ],
        scratch_shapes=[pltpu.VMEM((n, nhid), jnp.float32)],
        compiler_params=pltpu.CompilerParams(
            dimension_semantics=("arbitrary",)),
    )(adj, x, W, b.reshape(1, nhid), mu)
    return z, q
